# baseline (speedup 1.0000x reference)
import functools

import jax
import jax.numpy as jnp
from jax import lax
from jax.experimental import pallas as pl
from jax.experimental.pallas import tpu as pltpu
from jax.experimental.pallas import tpu_sc as plsc

_NUM_ROWS = 32
_NUM_COLS = 32
_EMBED_DIM = 768
_LANES = 16
_CHUNKS = _EMBED_DIM // _LANES

_mesh = plsc.VectorSubcoreMesh(core_axis_name="c", subcore_axis_name="s")


@functools.partial(
    pl.kernel,
    mesh=_mesh,
    out_type=jax.ShapeDtypeStruct((_NUM_ROWS * _NUM_COLS, _EMBED_DIM), jnp.float32),
    scratch_types=[
        pltpu.VMEM((_EMBED_DIM,), jnp.float32),
        pltpu.VMEM((_NUM_COLS, _EMBED_DIM), jnp.float32),
    ],
)
def _pos2d(wrow_hbm, wcol_hbm, out_hbm, wr_v, out_v):
    num_cores = 2
    wid = lax.axis_index("s") * num_cores + lax.axis_index("c")
    pltpu.sync_copy(wrow_hbm.at[pl.ds(wid * _EMBED_DIM, _EMBED_DIM)], wr_v)
    pltpu.sync_copy(wcol_hbm, out_v)

    def col_body(c, _):
        def chunk_body(j, _):
            sl = pl.ds(j * _LANES, _LANES)
            out_v[c, sl] = out_v[c, sl] + wr_v[sl]
            return 0

        return lax.fori_loop(0, _CHUNKS, chunk_body, 0)

    lax.fori_loop(0, _NUM_COLS, col_body, 0)
    pltpu.sync_copy(out_v, out_hbm.at[pl.ds(wid * _NUM_COLS, _NUM_COLS)])


def kernel(input, W_row, W_col):
    del input
    wr = W_row[1 : 1 + _NUM_ROWS].reshape(_NUM_ROWS * _EMBED_DIM)
    wc = W_col[1 : 1 + _NUM_COLS]
    out = _pos2d(wr, wc)
    return out.reshape(1, _NUM_ROWS * _NUM_COLS, _EMBED_DIM)


# speedup vs baseline: 1.2790x; 1.2790x over previous
import functools

import jax
import jax.numpy as jnp
from jax import lax
from jax.experimental import pallas as pl
from jax.experimental.pallas import tpu as pltpu
from jax.experimental.pallas import tpu_sc as plsc

_NUM_ROWS = 32
_NUM_COLS = 32
_EMBED_DIM = 768
_LANES = 16
_CHUNKS = _EMBED_DIM // _LANES

_mesh = plsc.VectorSubcoreMesh(core_axis_name="c", subcore_axis_name="s")


@functools.partial(
    pl.kernel,
    mesh=_mesh,
    out_type=jax.ShapeDtypeStruct((_NUM_ROWS * _NUM_COLS, _EMBED_DIM), jnp.float32),
    scratch_types=[
        pltpu.VMEM((_EMBED_DIM,), jnp.float32),
        pltpu.VMEM((_NUM_COLS, _EMBED_DIM), jnp.float32),
    ],
)
def _pos2d(wrow_hbm, wcol_hbm, out_hbm, wr_v, out_v):
    num_cores = 2
    wid = lax.axis_index("s") * num_cores + lax.axis_index("c")
    pltpu.sync_copy(wrow_hbm.at[pl.ds(wid * _EMBED_DIM, _EMBED_DIM)], wr_v)
    pltpu.sync_copy(wcol_hbm, out_v)

    wr_regs = [wr_v[pl.ds(j * _LANES, _LANES)] for j in range(_CHUNKS)]

    def col_body(c, _):
        for j in range(_CHUNKS):
            plsc.addupdate(out_v.at[c, pl.ds(j * _LANES, _LANES)], wr_regs[j])
        return 0

    lax.fori_loop(0, _NUM_COLS, col_body, 0)
    pltpu.sync_copy(out_v, out_hbm.at[pl.ds(wid * _NUM_COLS, _NUM_COLS)])


def kernel(input, W_row, W_col):
    del input
    wr = W_row[1 : 1 + _NUM_ROWS].reshape(_NUM_ROWS * _EMBED_DIM)
    wc = W_col[1 : 1 + _NUM_COLS]
    out = _pos2d(wr, wc)
    return out.reshape(1, _NUM_ROWS * _NUM_COLS, _EMBED_DIM)
